# baseline (speedup 1.0000x reference)
import jax
import jax.numpy as jnp
from jax import lax
from jax.experimental import pallas as pl
from jax.experimental.pallas import tpu as pltpu
from jax.experimental.pallas import tpu_sc as plsc

N = 10000
E = 320000
IN = 128
HID = 256
HALF = 64
NT = 16
NSUB = 16
CH = 80
CPT = E // (2 * NSUB * CH)
NB = 25
BN = N // NB


def _tc_pre_body(x_ref, emb_ref, wt1_ref, bt1_ref, wt2_ref, bt2_ref,
                 ws1_ref, bs1_ref, ws2_ref, bs2_ref, z_ref, sf_ref):
    x = x_ref[...]
    wt1 = wt1_ref[...]
    wt2 = wt2_ref[...]
    bt2 = bt2_ref[...]
    y = jnp.dot(x, wt1[IN:, :], preferred_element_type=jnp.float32)
    t = (jnp.dot(emb_ref[...], wt1[:IN, :],
                 preferred_element_type=jnp.float32) + bt1_ref[...])
    for ti in range(NT):
        r = jnp.maximum(y + t[ti:ti + 1, :], 0.0)
        z_ref[ti, :, :] = (jnp.dot(r, wt2,
                                   preferred_element_type=jnp.float32) + bt2)
    h = jnp.maximum(jnp.dot(x, ws1_ref[...],
                            preferred_element_type=jnp.float32)
                    + bs1_ref[...], 0.0)
    sf_ref[...] = (jnp.dot(h, ws2_ref[...],
                           preferred_element_type=jnp.float32) + bs2_ref[...])


def _tc_pre(x, emb, Wt1, bt1, Wt2, bt2, Ws1, bs1, Ws2, bs2):
    return pl.pallas_call(
        _tc_pre_body,
        grid=(NB,),
        in_specs=[
            pl.BlockSpec((BN, IN), lambda i: (i, 0)),
            pl.BlockSpec((NT, IN), lambda i: (0, 0)),
            pl.BlockSpec((HID, HID), lambda i: (0, 0)),
            pl.BlockSpec((1, HID), lambda i: (0, 0)),
            pl.BlockSpec((HID, HALF), lambda i: (0, 0)),
            pl.BlockSpec((1, HALF), lambda i: (0, 0)),
            pl.BlockSpec((IN, IN), lambda i: (0, 0)),
            pl.BlockSpec((1, IN), lambda i: (0, 0)),
            pl.BlockSpec((IN, HALF), lambda i: (0, 0)),
            pl.BlockSpec((1, HALF), lambda i: (0, 0)),
        ],
        out_specs=[
            pl.BlockSpec((NT, BN, HALF), lambda i: (0, i, 0)),
            pl.BlockSpec((BN, HALF), lambda i: (i, 0)),
        ],
        out_shape=[
            jax.ShapeDtypeStruct((NT, N, HALF), jnp.float32),
            jax.ShapeDtypeStruct((N, HALF), jnp.float32),
        ],
    )(x, emb, Wt1, bt1.reshape(1, HID), Wt2, bt2.reshape(1, HALF),
      Ws1, bs1.reshape(1, IN), Ws2, bs2.reshape(1, HALF))


def _sc_edges_body(eidx_h, z_h,
                   h0_out, h1_out,
                   h_sh, idx0, idx1, sb0, sb1, rows0, rows1,
                   isem0, isem1, gsem0, gsem1, ssem0, ssem1):
    c = lax.axis_index("c")
    s = lax.axis_index("s")

    zero16 = jnp.zeros((16,), jnp.float32)
    rows = (rows0, rows1)
    idxb = (idx0, idx1)
    sbuf = (sb0, sb1)
    isem = (isem0, isem1)
    gsem = (gsem0, gsem1)
    ssem = (ssem0, ssem1)

    def z_row(i, carry):
        for j in range(4):
            rows0[i, pl.ds(j * 16, 16)] = zero16
        return carry
    lax.fori_loop(0, CH, z_row, 0)

    r0 = s * 624
    for k in range(7):
        pltpu.sync_copy(rows0, h_sh.at[pl.ds(r0 + k * CH, CH)])

    @pl.when(s < 15)
    def _():
        pltpu.sync_copy(rows0.at[pl.ds(0, 64)], h_sh.at[pl.ds(r0 + 560, 64)])

    @pl.when(s == 15)
    def _():
        pltpu.sync_copy(rows0, h_sh.at[pl.ds(r0 + 560, CH)])

    plsc.subcore_barrier()

    row_t = c * (NSUB * CPT) + s * CPT

    def idx_desc(b, il):
        return pltpu.make_async_copy(eidx_h.at[row_t + il], idxb[b], isem[b])

    def gather_descs(b):
        return [
            pltpu.make_async_copy(z_h.at[idxb[b].at[0, pl.ds(0, 40)]],
                                  rows[b].at[pl.ds(0, 40)], gsem[b]),
            pltpu.make_async_copy(z_h.at[idxb[b].at[0, pl.ds(40, 40)]],
                                  rows[b].at[pl.ds(40, 40)], gsem[b]),
        ]

    def gather_start(b):
        for d in gather_descs(b):
            d.start()

    def gather_wait(b):
        for d in gather_descs(b):
            d.wait()

    def scat_desc(b):
        return pltpu.make_async_copy(rows[b], h_sh.at[sbuf[b].at[0]], ssem[b])

    def stash(b):
        for j in range(5):
            sl = pl.ds(j * 16, 16)
            sbuf[b][0, sl] = idxb[b][1, sl]

    def scatter(b):
        pltpu.async_copy(rows[b], h_sh.at[sbuf[b].at[0]], ssem[b], add=True)

    pltpu.sync_copy(eidx_h.at[row_t], idx0)
    idx_desc(1, 1).start()
    gather_start(0)

    def pair(g, carry):
        @pl.when(g > 0)
        def _():
            scat_desc(1).wait()
        idx_desc(1, 0).wait()
        gather_start(1)
        gather_wait(0)
        stash(0)
        idx_desc(0, 2 * g + 2).start()
        scatter(0)

        scat_desc(0).wait()
        idx_desc(0, 0).wait()
        gather_start(0)
        gather_wait(1)
        stash(1)

        @pl.when(g < CPT // 2 - 1)
        def _():
            idx_desc(1, 2 * g + 3).start()
        scatter(1)
        return carry

    lax.fori_loop(0, CPT // 2, pair, 0)

    scat_desc(1).wait()
    gather_wait(0)
    stash(0)
    scatter(0)
    scat_desc(0).wait()
    plsc.subcore_barrier()

    def stripe_out(h_out):
        for k in range(7):
            pltpu.sync_copy(h_sh.at[pl.ds(r0 + k * CH, CH)], rows0)
            pltpu.sync_copy(rows0, h_out.at[pl.ds(r0 + k * CH, CH)])

        @pl.when(s < 15)
        def _():
            pltpu.sync_copy(h_sh.at[pl.ds(r0 + 560, 64)],
                            rows0.at[pl.ds(0, 64)])
            pltpu.sync_copy(rows0.at[pl.ds(0, 64)],
                            h_out.at[pl.ds(r0 + 560, 64)])

        @pl.when(s == 15)
        def _():
            pltpu.sync_copy(h_sh.at[pl.ds(r0 + 560, CH)], rows0)
            pltpu.sync_copy(rows0, h_out.at[pl.ds(r0 + 560, CH)])

    @pl.when(c == 0)
    def _():
        stripe_out(h0_out)

    @pl.when(c == 1)
    def _():
        stripe_out(h1_out)


def _sc_edges(cidx, dst, z):
    eidx = jnp.stack([cidx.reshape(E // CH, CH), dst.reshape(E // CH, CH)],
                     axis=1)
    mesh = plsc.VectorSubcoreMesh(core_axis_name="c", subcore_axis_name="s")
    f = pl.kernel(
        _sc_edges_body,
        out_type=[
            jax.ShapeDtypeStruct((N, HALF), jnp.float32),
            jax.ShapeDtypeStruct((N, HALF), jnp.float32),
        ],
        mesh=mesh,
        scratch_types=[
            pltpu.VMEM_SHARED((N, HALF), jnp.float32),
            pltpu.VMEM((2, CH), jnp.int32),
            pltpu.VMEM((2, CH), jnp.int32),
            pltpu.VMEM((1, CH), jnp.int32),
            pltpu.VMEM((1, CH), jnp.int32),
            pltpu.VMEM((CH, HALF), jnp.float32),
            pltpu.VMEM((CH, HALF), jnp.float32),
            pltpu.SemaphoreType.DMA,
            pltpu.SemaphoreType.DMA,
            pltpu.SemaphoreType.DMA,
            pltpu.SemaphoreType.DMA,
            pltpu.SemaphoreType.DMA,
            pltpu.SemaphoreType.DMA,
        ],
        compiler_params=pltpu.CompilerParams(use_tc_tiling_on_sc=False,
                                             needs_layout_passes=False),
    )
    return f(eidx, z.reshape(NT * N, HALF))


def _tc_post_body(h0_ref, h1_ref, sf_ref, out_ref):
    msg = h0_ref[...] + h1_ref[...]
    out_ref[...] = jnp.concatenate([sf_ref[...], msg], axis=-1)


def _tc_post(h0, h1, sf):
    return pl.pallas_call(
        _tc_post_body,
        out_shape=jax.ShapeDtypeStruct((N, 2 * HALF), jnp.float32),
    )(h0, h1, sf)


def kernel(x, edge_index, edge_type, emb, Ws1, bs1, Ws2, bs2,
           Wt1, bt1, Wt2, bt2):
    src = edge_index[0].astype(jnp.int32)
    dst = edge_index[1].astype(jnp.int32)
    et = edge_type.astype(jnp.int32)
    cidx = et * N + src

    z, sf = _tc_pre(x, emb, Wt1, bt1, Wt2, bt2, Ws1, bs1, Ws2, bs2)
    h0, h1 = _sc_edges(cidx, dst, z)
    return _tc_post(h0, h1, sf)
